# Initial kernel scaffold; baseline (speedup 1.0000x reference)
import jax
import jax.numpy as jnp
from jax.experimental import pallas as pl


def kernel(pred, target, label_weight):
    raise NotImplementedError("write your pallas kernel here")


# speedup vs baseline: 3.1744x; 3.1744x over previous
import functools

import numpy as np
import jax
import jax.numpy as jnp
from jax import lax
from jax.experimental import pallas as pl
from jax.experimental.pallas import tpu as pltpu
from jax.experimental.pallas import tpu_sc as plsc

_BINS = 10
_NC, _NS = 2, 16
_NW = _NC * _NS
_B, _C = 100000, 80
_N = _B * _C
_PER_W = _N // _NW
_CH = 10000
_NCHUNK = _PER_W // _CH
_VIT = _CH // 16
_TBL = 2 * 11 * 16

_EDGES = [float(np.float32(k) / np.float32(10.0)) for k in range(11)]


def _sc_body(pred_hbm, targ_hbm, lw_hbm, out_hbm, pv, tv, lv, acc, s1, s2, s3):
    wid = lax.axis_index("c") * _NS + lax.axis_index("s")
    base = wid * _PER_W
    zeros16 = jnp.zeros((16,), jnp.float32)
    for j in range(_TBL // 16):
        acc[pl.ds(j * 16, 16)] = zeros16
    lane = lax.iota(jnp.int32, 16)
    ones16 = jnp.full((16,), 1.0, jnp.float32)

    def chunk_body(c, carry):
        off = base + c * _CH
        cp1 = pltpu.async_copy(pred_hbm.at[pl.ds(off, _CH)], pv, s1)
        cp2 = pltpu.async_copy(targ_hbm.at[pl.ds(off, _CH)], tv, s2)
        cp3 = pltpu.async_copy(lw_hbm.at[pl.ds(off, _CH)], lv, s3)
        cp1.wait()
        cp2.wait()
        cp3.wait()

        def vec_body(i, c2):
            o = i * 16
            p = pv[pl.ds(o, 16)]
            t = tv[pl.ds(o, 16)].astype(jnp.float32)
            w = lv[pl.ds(o, 16)]
            z = jnp.exp(-jnp.abs(p))
            sig = jnp.where(p >= 0.0, 1.0, z) / (1.0 + z)
            g = jnp.abs(sig - t)
            u = z / (z + 2.0)
            u2 = u * u
            log1pz = (2.0 * u) * (1.0 + u2 * (
                0.3333333333 + u2 * (0.2 + u2 * (0.1428571429 + u2 * 0.1111111111))))
            loss_el = jnp.maximum(p, 0.0) - p * t + log1pz
            valid = w > 0.0
            idx = jnp.zeros((16,), jnp.int32)
            for k in range(1, 10):
                idx = idx + jnp.where(g >= _EDGES[k], 1, 0)
            idx = jnp.where(valid, idx, _BINS)
            sidx = idx * 16 + lane
            plsc.addupdate_scatter(acc, [sidx], jnp.where(valid, loss_el, 0.0))
            plsc.addupdate_scatter(acc, [sidx + 176], ones16)
            return c2

        lax.fori_loop(0, _VIT, vec_body, 0)
        return carry

    lax.fori_loop(0, _NCHUNK, chunk_body, 0)
    pltpu.sync_copy(acc, out_hbm.at[wid])


_sc_pass = functools.partial(
    pl.kernel,
    mesh=plsc.VectorSubcoreMesh(core_axis_name="c", subcore_axis_name="s"),
    out_type=jax.ShapeDtypeStruct((_NW, _TBL), jnp.float32),
    compiler_params=pltpu.CompilerParams(needs_layout_passes=False),
    scratch_types=[
        pltpu.VMEM((_CH,), jnp.float32),
        pltpu.VMEM((_CH,), jnp.int32),
        pltpu.VMEM((_CH,), jnp.float32),
        pltpu.VMEM((_TBL,), jnp.float32),
        pltpu.SemaphoreType.DMA,
        pltpu.SemaphoreType.DMA,
        pltpu.SemaphoreType.DMA,
    ],
)(_sc_body)


def _finish_body(part_ref, out_ref):
    part = part_ref[...]
    col = jnp.sum(part, axis=0, keepdims=True)
    s_tot = jnp.float32(0.0)
    n = jnp.float32(0.0)
    for b in range(_BINS):
        sb = jnp.sum(col[:, 16 * b:16 * (b + 1)])
        cb = jnp.sum(col[:, 176 + 16 * b:176 + 16 * (b + 1)])
        ne = cb > 0.0
        s_tot = s_tot + jnp.where(ne, sb / jnp.maximum(cb, 1.0), 0.0)
        n = n + jnp.where(ne, 1.0, 0.0)
    out_ref[0, 0] = jnp.where(n > 0.0, s_tot / n, 0.0)


_finish = pl.pallas_call(
    _finish_body,
    out_shape=jax.ShapeDtypeStruct((1, 1), jnp.float32),
    out_specs=pl.BlockSpec(memory_space=pltpu.SMEM),
)


def kernel(pred, target, label_weight):
    p = pred.reshape(-1)
    t = target.reshape(-1)
    w = label_weight.reshape(-1)
    part = _sc_pass(p, t, w)
    return _finish(part)[0, 0]
